# baseline (speedup 1.0000x reference)
import jax
import jax.numpy as jnp
from jax import lax
from jax.experimental import pallas as pl
from jax.experimental.pallas import tpu as pltpu
from jax.experimental.pallas import tpu_sc as plsc

EMBED = 32
OUT_D = 96
N_ROWS = 2 * 7 * 21
TAB_W = 512
BATCH = 16384
NC, NS = 2, 16
NW = NC * NS
NFG = 4
NBB = NW // NFG
FPW = OUT_D // NFG
BPW = BATCH // NBB
CHUNK = 128
NCH = BPW // CHUNK
L = 16


def _body(xg, xa, xo, tab, out, idx_v, fused_v, tab_v, out_v):
    c = lax.axis_index("c")
    s = lax.axis_index("s")
    wid = s * NC + c
    fg = wid % NFG
    bb = wid // NFG
    rbase = bb * NCH

    pltpu.sync_copy(tab.at[pl.ds(fg * FPW * TAB_W, FPW * TAB_W)], tab_v)
    pltpu.sync_copy(xg.at[pl.ds(rbase, NCH)], idx_v.at[0])
    pltpu.sync_copy(xa.at[pl.ds(rbase, NCH)], idx_v.at[1])
    pltpu.sync_copy(xo.at[pl.ds(rbase, NCH)], idx_v.at[2])

    for j in range(NCH):
        for i in range(CHUNK // L):
            sl = pl.ds(i * L, L)
            g = idx_v[0, j, sl]
            a = idx_v[1, j, sl]
            o = idx_v[2, j, sl]
            fused_v[j, sl] = g * 147 + a * 21 + o

    def _lookup(j):
        for i in range(CHUNK // L):
            idx16 = fused_v[j, pl.ds(i * L, L)]
            for f in range(FPW):
                out_v[f, pl.ds(j * CHUNK + i * L, L)] = plsc.load_gather(
                    tab_v, [idx16 + (f * TAB_W)])

    pl.loop(0, NCH)(_lookup)

    pltpu.sync_copy(out_v, out.at[pl.ds(fg * FPW, FPW), pl.ds(bb * BPW, BPW)])


def kernel(x1, W_gender, W_age, W_occupation):
    xg = x1[:, 0].reshape(BATCH // CHUNK, CHUNK)
    xa = x1[:, 1].reshape(BATCH // CHUNK, CHUNK)
    xo = x1[:, 2].reshape(BATCH // CHUNK, CHUNK)
    wcat = jnp.concatenate(
        [
            jnp.broadcast_to(W_gender[:, None, None, :], (2, 7, 21, EMBED)),
            jnp.broadcast_to(W_age[None, :, None, :], (2, 7, 21, EMBED)),
            jnp.broadcast_to(W_occupation[None, None, :, :], (2, 7, 21, EMBED)),
        ],
        axis=-1,
    ).reshape(N_ROWS, OUT_D)
    tab = jnp.pad(wcat.T, ((0, 0), (0, TAB_W - N_ROWS))).reshape(OUT_D * TAB_W)
    k = pl.kernel(
        _body,
        out_type=jax.ShapeDtypeStruct((OUT_D, BATCH), jnp.float32),
        mesh=plsc.VectorSubcoreMesh(core_axis_name="c", subcore_axis_name="s"),
        scratch_types=[
            pltpu.VMEM((3, NCH, CHUNK), jnp.int32),
            pltpu.VMEM((NCH, CHUNK), jnp.int32),
            pltpu.VMEM((FPW * TAB_W,), jnp.float32),
            pltpu.VMEM((FPW, BPW), jnp.float32),
        ],
        compiler_params=pltpu.CompilerParams(needs_layout_passes=False),
    )
    return k(xg, xa, xo, tab).T
